# baseline (speedup 1.0000x reference)
import functools

import jax
import jax.numpy as jnp
from jax import lax
from jax.experimental import pallas as pl
from jax.experimental.pallas import tpu as pltpu
from jax.experimental.pallas import tpu_sc as plsc

_T, _D, _E, _H = 2048, 768, 64, 1024
_C = 64
_S = _E * _C
_NC, _NS = 2, 16
_NW = _NC * _NS


def _router_body(x_ref, rw_ref, slot_ref, gain_ref):
    logits = jnp.dot(x_ref[...], rw_ref[...], preferred_element_type=jnp.float32)
    m = jnp.max(logits, axis=-1, keepdims=True)
    ex = jnp.exp(logits - m)
    s = jnp.sum(ex, axis=-1, keepdims=True)
    probs = ex / s
    pm = jnp.max(probs, axis=-1, keepdims=True)
    iota_e = lax.broadcasted_iota(jnp.int32, (_T, _E), 1)
    e = jnp.min(jnp.where(probs == pm, iota_e, _E), axis=-1, keepdims=True)
    onehot = (iota_e == e).astype(jnp.float32)

    chunk = 256
    r = lax.broadcasted_iota(jnp.int32, (chunk, chunk), 0)
    c = lax.broadcasted_iota(jnp.int32, (chunk, chunk), 1)
    tri = (r > c).astype(jnp.float32)
    counts = jnp.zeros((1, _E), jnp.float32)
    pos_cols = []
    for i in range(_T // chunk):
        oh = onehot[i * chunk:(i + 1) * chunk, :]
        pos_chunk = jnp.dot(tri, oh, preferred_element_type=jnp.float32) + counts
        pos_cols.append(jnp.sum(pos_chunk * oh, axis=-1, keepdims=True))
        counts = counts + jnp.sum(oh, axis=0, keepdims=True)
    pos = jnp.concatenate(pos_cols, axis=0).astype(jnp.int32)

    keep = pos < _C
    slot_ref[...] = jnp.where(keep, e * _C + pos, -1)
    gain_ref[...] = jnp.where(keep, pm, 0.0)


def _router(x, router_w):
    return pl.pallas_call(
        _router_body,
        out_shape=(
            jax.ShapeDtypeStruct((_T, 1), jnp.int32),
            jax.ShapeDtypeStruct((_T, 1), jnp.float32),
        ),
    )(x, router_w)


_G = 2
_SG = _S // _G
_SPW = _SG // _NW
_SSC = _SG // _NC
_TPW = _T // _NS


def _dispatch_body(g, slot_hbm, x_hbm, ein_hbm,
                   slots_v, idx_v, val_v, tok_v, rows_v, shared_tok, sem):
    cid = lax.axis_index("c")
    sid = lax.axis_index("s")
    sc_base = g * _SG + cid * _SSC

    dbase = (sc_base + sid * _SPW) & (_T - 1)
    for i in range(_SPW // 16):
        tok_v[pl.ds(i * 16, 16)] = lax.broadcasted_iota(jnp.int32, (16,), 0) + (dbase + i * 16)
    pltpu.sync_copy(tok_v, shared_tok.at[pl.ds(sid * _SPW, _SPW)])
    plsc.subcore_barrier()

    tbase = sid * _TPW
    pltpu.sync_copy(slot_hbm.at[pl.ds(tbase, _TPW)], slots_v)
    for i in range(_TPW // 16):
        sl = slots_v[pl.ds(i * 16, 16)]
        tok = lax.broadcasted_iota(jnp.int32, (16,), 0) + (tbase + i * 16)
        valid = (sl >= sc_base) & (sl < sc_base + _SSC)
        idx_v[pl.ds(i * 16, 16)] = jnp.where(valid, sl - sc_base, _SSC + sid)
        val_v[pl.ds(i * 16, 16)] = tok
    pltpu.sync_copy(val_v, shared_tok.at[idx_v])
    plsc.subcore_barrier()

    pltpu.sync_copy(shared_tok.at[pl.ds(sid * _SPW, _SPW)], tok_v)
    pltpu.async_copy(x_hbm.at[tok_v], rows_v, sem).wait()
    pltpu.sync_copy(rows_v, ein_hbm.at[pl.ds(cid * _SSC + sid * _SPW, _SPW)])


def _dispatch(slot, x, g):
    mesh = plsc.VectorSubcoreMesh(core_axis_name="c", subcore_axis_name="s",
                                   num_cores=_NC, num_subcores=_NS)
    return pl.kernel(
        functools.partial(_dispatch_body, g),
        mesh=mesh,
        compiler_params=pltpu.CompilerParams(needs_layout_passes=False),
        out_type=jax.ShapeDtypeStruct((_SG, _D), jnp.float32),
        scratch_types=[
            pltpu.VMEM((_TPW,), jnp.int32),
            pltpu.VMEM((_TPW,), jnp.int32),
            pltpu.VMEM((_TPW,), jnp.int32),
            pltpu.VMEM((_SPW,), jnp.int32),
            pltpu.VMEM((_SPW, _D), jnp.float32),
            pltpu.VMEM_SHARED((_SSC + _NS,), jnp.int32),
            pltpu.SemaphoreType.DMA,
        ],
    )(slot, x)


_EB = 2


def _mlp_half_body(ein_ref, w1_ref, w2_ref, out_ref):
    for k in range(_EB):
        a = ein_ref[pl.ds(k * _C, _C), :]
        h = jnp.maximum(
            jnp.dot(a, w1_ref[k], preferred_element_type=jnp.float32), 0.0)
        out_ref[pl.ds(k * _C, _C), :] = jnp.dot(
            h, w2_ref[k], preferred_element_type=jnp.float32)


def _mlp_half(ein_g, w1, w2, eout_prev, g):
    eg = _E // _G
    base = g * eg // _EB
    specs = [
        pl.BlockSpec((_EB * _C, _D), lambda e: (e, 0)),
        pl.BlockSpec((_EB, _D, _H), lambda e, b=base: (e + b, 0, 0)),
        pl.BlockSpec((_EB, _H, _D), lambda e, b=base: (e + b, 0, 0)),
    ]
    args = [ein_g, w1, w2]
    aliases = {}
    body = _mlp_half_body
    if eout_prev is not None:
        specs.append(pl.BlockSpec((_EB * _C, _D), lambda e: (0, 0)))
        args.append(eout_prev)
        aliases = {3: 0}
        body = lambda a, b, c, prev, o: _mlp_half_body(a, b, c, o)
    return pl.pallas_call(
        body,
        grid=(eg // _EB,),
        in_specs=specs,
        out_specs=pl.BlockSpec((_EB * _C, _D), lambda e, b=base: (e + b, 0)),
        out_shape=jax.ShapeDtypeStruct((_S, _D), jnp.float32),
        input_output_aliases=aliases,
        compiler_params=pltpu.CompilerParams(
            dimension_semantics=("arbitrary",),
            vmem_limit_bytes=100 * 1024 * 1024,
        ),
    )(*args)


def _combine_body(slot_hbm, gain_hbm, eout_hbm, out_hbm, slot_v, gain_v, rows_v, sem):
    wid = lax.axis_index("s") * _NC + lax.axis_index("c")
    tpw = _T // _NW
    base = wid * tpw
    pltpu.sync_copy(slot_hbm.at[pl.ds(base, tpw)], slot_v)
    pltpu.sync_copy(gain_hbm.at[pl.ds(base, tpw)], gain_v)
    for i in range(tpw // 16):
        s = slot_v[pl.ds(i * 16, 16)]
        slot_v[pl.ds(i * 16, 16)] = jnp.maximum(s, 0)
    pltpu.async_copy(eout_hbm.at[slot_v], rows_v, sem).wait()

    def scale(j, carry):
        g = plsc.load_gather(gain_v, [jnp.full((16,), j, jnp.int32)])
        for cc in range(_D // 16):
            rows_v[j, pl.ds(cc * 16, 16)] = rows_v[j, pl.ds(cc * 16, 16)] * g
        return carry

    lax.fori_loop(0, tpw, scale, 0)
    pltpu.sync_copy(rows_v, out_hbm.at[pl.ds(base, tpw)])


def _combine(slot, gain, eout):
    mesh = plsc.VectorSubcoreMesh(core_axis_name="c", subcore_axis_name="s",
                                   num_cores=_NC, num_subcores=_NS)
    return pl.kernel(
        _combine_body,
        mesh=mesh,
        compiler_params=pltpu.CompilerParams(needs_layout_passes=False),
        out_type=jax.ShapeDtypeStruct((_T, _D), jnp.float32),
        scratch_types=[
            pltpu.VMEM((_T // _NW,), jnp.int32),
            pltpu.VMEM((_T // _NW,), jnp.float32),
            pltpu.VMEM((_T // _NW, _D), jnp.float32),
            pltpu.SemaphoreType.DMA,
        ],
    )(slot, gain, eout)


def kernel(x, router_w, w1, w2):
    slot2d, gain2d = _router(x, router_w)
    slot = slot2d.reshape(_T)
    gain = gain2d.reshape(_T)
    eins = [_dispatch(slot, x, g) for g in range(_G)]
    eout = None
    for g in range(_G):
        eout = _mlp_half(eins[g], w1, w2, eout, g)
    return _combine(slot, gain, eout)


# speedup vs baseline: 1.0205x; 1.0205x over previous
import functools

import jax
import jax.numpy as jnp
from jax import lax
from jax.experimental import pallas as pl
from jax.experimental.pallas import tpu as pltpu
from jax.experimental.pallas import tpu_sc as plsc

_T, _D, _E, _H = 2048, 768, 64, 1024
_C = 64
_S = _E * _C
_NC, _NS = 2, 16
_NW = _NC * _NS


def _router_body(x_ref, rw_ref, slot_ref, gain_ref):
    logits = jnp.dot(x_ref[...], rw_ref[...], preferred_element_type=jnp.float32)
    m = jnp.max(logits, axis=-1, keepdims=True)
    ex = jnp.exp(logits - m)
    s = jnp.sum(ex, axis=-1, keepdims=True)
    probs = ex / s
    pm = jnp.max(probs, axis=-1, keepdims=True)
    iota_e = lax.broadcasted_iota(jnp.int32, (_T, _E), 1)
    e = jnp.min(jnp.where(probs == pm, iota_e, _E), axis=-1, keepdims=True)
    onehot = (iota_e == e).astype(jnp.float32)

    chunk = 256
    r = lax.broadcasted_iota(jnp.int32, (chunk, chunk), 0)
    c = lax.broadcasted_iota(jnp.int32, (chunk, chunk), 1)
    tri = (r > c).astype(jnp.float32)
    counts = jnp.zeros((1, _E), jnp.float32)
    pos_cols = []
    for i in range(_T // chunk):
        oh = onehot[i * chunk:(i + 1) * chunk, :]
        pos_chunk = jnp.dot(tri, oh, preferred_element_type=jnp.float32) + counts
        pos_cols.append(jnp.sum(pos_chunk * oh, axis=-1, keepdims=True))
        counts = counts + jnp.sum(oh, axis=0, keepdims=True)
    pos = jnp.concatenate(pos_cols, axis=0).astype(jnp.int32)

    keep = pos < _C
    slot_ref[...] = jnp.where(keep, e * _C + pos, -1)
    gain_ref[...] = jnp.where(keep, pm, 0.0)


def _router(x, router_w):
    return pl.pallas_call(
        _router_body,
        out_shape=(
            jax.ShapeDtypeStruct((_T, 1), jnp.int32),
            jax.ShapeDtypeStruct((_T, 1), jnp.float32),
        ),
    )(x, router_w)


_G = 1
_SG = _S // _G
_SPW = _SG // _NW
_SSC = _SG // _NC
_TPW = _T // _NS


def _dispatch_body(g, slot_hbm, x_hbm, ein_hbm,
                   slots_v, idx_v, val_v, tok_v, rows_v, shared_tok, sem):
    cid = lax.axis_index("c")
    sid = lax.axis_index("s")
    sc_base = g * _SG + cid * _SSC

    dbase = (sc_base + sid * _SPW) & (_T - 1)
    for i in range(_SPW // 16):
        tok_v[pl.ds(i * 16, 16)] = lax.broadcasted_iota(jnp.int32, (16,), 0) + (dbase + i * 16)
    pltpu.sync_copy(tok_v, shared_tok.at[pl.ds(sid * _SPW, _SPW)])
    plsc.subcore_barrier()

    tbase = sid * _TPW
    pltpu.sync_copy(slot_hbm.at[pl.ds(tbase, _TPW)], slots_v)
    for i in range(_TPW // 16):
        sl = slots_v[pl.ds(i * 16, 16)]
        tok = lax.broadcasted_iota(jnp.int32, (16,), 0) + (tbase + i * 16)
        valid = (sl >= sc_base) & (sl < sc_base + _SSC)
        idx_v[pl.ds(i * 16, 16)] = jnp.where(valid, sl - sc_base, _SSC + sid)
        val_v[pl.ds(i * 16, 16)] = tok
    pltpu.sync_copy(val_v, shared_tok.at[idx_v])
    plsc.subcore_barrier()

    pltpu.sync_copy(shared_tok.at[pl.ds(sid * _SPW, _SPW)], tok_v)
    pltpu.async_copy(x_hbm.at[tok_v], rows_v, sem).wait()
    pltpu.sync_copy(rows_v, ein_hbm.at[pl.ds(cid * _SSC + sid * _SPW, _SPW)])


def _dispatch(slot, x, g):
    mesh = plsc.VectorSubcoreMesh(core_axis_name="c", subcore_axis_name="s",
                                   num_cores=_NC, num_subcores=_NS)
    return pl.kernel(
        functools.partial(_dispatch_body, g),
        mesh=mesh,
        compiler_params=pltpu.CompilerParams(needs_layout_passes=False),
        out_type=jax.ShapeDtypeStruct((_SG, _D), jnp.float32),
        scratch_types=[
            pltpu.VMEM((_TPW,), jnp.int32),
            pltpu.VMEM((_TPW,), jnp.int32),
            pltpu.VMEM((_TPW,), jnp.int32),
            pltpu.VMEM((_SPW,), jnp.int32),
            pltpu.VMEM((_SPW, _D), jnp.float32),
            pltpu.VMEM_SHARED((_SSC + _NS,), jnp.int32),
            pltpu.SemaphoreType.DMA,
        ],
    )(slot, x)


_EB = 2


def _mlp_half_body(ein_ref, w1_ref, w2_ref, out_ref):
    for k in range(_EB):
        a = ein_ref[pl.ds(k * _C, _C), :]
        h = jnp.maximum(
            jnp.dot(a, w1_ref[k], preferred_element_type=jnp.float32), 0.0)
        out_ref[pl.ds(k * _C, _C), :] = jnp.dot(
            h, w2_ref[k], preferred_element_type=jnp.float32)


def _mlp_half(ein_g, w1, w2, eout_prev, g):
    eg = _E // _G
    base = g * eg // _EB
    specs = [
        pl.BlockSpec((_EB * _C, _D), lambda e: (e, 0)),
        pl.BlockSpec((_EB, _D, _H), lambda e, b=base: (e + b, 0, 0)),
        pl.BlockSpec((_EB, _H, _D), lambda e, b=base: (e + b, 0, 0)),
    ]
    args = [ein_g, w1, w2]
    aliases = {}
    body = _mlp_half_body
    if eout_prev is not None:
        specs.append(pl.BlockSpec((_EB * _C, _D), lambda e: (0, 0)))
        args.append(eout_prev)
        aliases = {3: 0}
        body = lambda a, b, c, prev, o: _mlp_half_body(a, b, c, o)
    return pl.pallas_call(
        body,
        grid=(eg // _EB,),
        in_specs=specs,
        out_specs=pl.BlockSpec((_EB * _C, _D), lambda e, b=base: (e + b, 0)),
        out_shape=jax.ShapeDtypeStruct((_S, _D), jnp.float32),
        input_output_aliases=aliases,
        compiler_params=pltpu.CompilerParams(
            dimension_semantics=("arbitrary",),
            vmem_limit_bytes=100 * 1024 * 1024,
        ),
    )(*args)


def _combine_body(slot_hbm, gain_hbm, eout_hbm, out_hbm, slot_v, gain_v, rows_v, sem):
    wid = lax.axis_index("s") * _NC + lax.axis_index("c")
    tpw = _T // _NW
    base = wid * tpw
    pltpu.sync_copy(slot_hbm.at[pl.ds(base, tpw)], slot_v)
    pltpu.sync_copy(gain_hbm.at[pl.ds(base, tpw)], gain_v)
    for i in range(tpw // 16):
        s = slot_v[pl.ds(i * 16, 16)]
        slot_v[pl.ds(i * 16, 16)] = jnp.maximum(s, 0)
    pltpu.async_copy(eout_hbm.at[slot_v], rows_v, sem).wait()

    def scale(j, carry):
        g = plsc.load_gather(gain_v, [jnp.full((16,), j, jnp.int32)])
        for cc in range(_D // 16):
            rows_v[j, pl.ds(cc * 16, 16)] = rows_v[j, pl.ds(cc * 16, 16)] * g
        return carry

    lax.fori_loop(0, tpw, scale, 0)
    pltpu.sync_copy(rows_v, out_hbm.at[pl.ds(base, tpw)])


def _combine(slot, gain, eout):
    mesh = plsc.VectorSubcoreMesh(core_axis_name="c", subcore_axis_name="s",
                                   num_cores=_NC, num_subcores=_NS)
    return pl.kernel(
        _combine_body,
        mesh=mesh,
        compiler_params=pltpu.CompilerParams(needs_layout_passes=False),
        out_type=jax.ShapeDtypeStruct((_T, _D), jnp.float32),
        scratch_types=[
            pltpu.VMEM((_T // _NW,), jnp.int32),
            pltpu.VMEM((_T // _NW,), jnp.float32),
            pltpu.VMEM((_T // _NW, _D), jnp.float32),
            pltpu.SemaphoreType.DMA,
        ],
    )(slot, gain, eout)


def kernel(x, router_w, w1, w2):
    slot2d, gain2d = _router(x, router_w)
    slot = slot2d.reshape(_T)
    gain = gain2d.reshape(_T)
    eins = [_dispatch(slot, x, g) for g in range(_G)]
    eout = None
    for g in range(_G):
        eout = _mlp_half(eins[g], w1, w2, eout, g)
    return _combine(slot, gain, eout)
